# baseline (speedup 1.0000x reference)
import functools

import jax
import jax.numpy as jnp
from jax import lax
from jax.experimental import pallas as pl
from jax.experimental.pallas import tpu as pltpu
from jax.experimental.pallas import tpu_sc as plsc

N = 10000
E = 320000
D = 128
H = 64
NC = 2
NS = 16
L = 16
CH = 80
NROWS = E // CH
NPT = 624
ZR = 16
F32 = jnp.float32


def _node_lin_body(h_ref, aw, ab, bw, bb, dw, db, ew, eb,
                   ah_o, bh_o, dh_o, eh_o):
    x = h_ref[...]
    ah_o[...] = jnp.dot(x, aw[...], preferred_element_type=F32) + ab[...]
    for w, b, o in ((bw, bb, bh_o), (dw, db, dh_o), (ew, eb, eh_o)):
        full = jnp.dot(x, w[...], preferred_element_type=F32) + b[...]
        o[0] = full[:, :H]
        o[1] = full[:, H:]


def _node_linears(h, A_w, A_b, B_w, B_b, D_w, D_b, E_w, E_b):
    blk = 2000
    grid = (N // blk,)
    wspec = pl.BlockSpec((D, D), lambda i: (0, 0))
    bspec = pl.BlockSpec((1, D), lambda i: (0, 0))
    half = pl.BlockSpec((2, blk, H), lambda i: (0, i, 0))
    return pl.pallas_call(
        _node_lin_body,
        grid=grid,
        in_specs=[pl.BlockSpec((blk, D), lambda i: (i, 0)),
                  wspec, bspec, wspec, bspec, wspec, bspec, wspec, bspec],
        out_specs=[pl.BlockSpec((blk, D), lambda i: (i, 0)), half, half, half],
        out_shape=[jax.ShapeDtypeStruct((N, D), F32),
                   jax.ShapeDtypeStruct((2, N, H), F32),
                   jax.ShapeDtypeStruct((2, N, H), F32),
                   jax.ShapeDtypeStruct((2, N, H), F32)],
    )(h, A_w, A_b, B_w, B_b, D_w, D_b, E_w, E_b)


def _ce_body(e_ref, cw, cb, ce_o):
    ce_o[...] = jnp.dot(e_ref[...], cw[...], preferred_element_type=F32) + cb[...]


def _ce_full(e, C_w, C_b):
    blk = 2560
    return pl.pallas_call(
        _ce_body,
        grid=(E // blk,),
        in_specs=[pl.BlockSpec((blk, D), lambda i: (i, 0)),
                  pl.BlockSpec((D, D), lambda i: (0, 0)),
                  pl.BlockSpec((1, D), lambda i: (0, 0))],
        out_specs=pl.BlockSpec((blk, D), lambda i: (i, 0)),
        out_shape=jax.ShapeDtypeStruct((E, D), F32),
    )(e, C_w, C_b)


NBUF = 3
NBLK = NROWS // NS


def _sc_edge_body(idx_hbm, dh_t, eh_t, bh_t, ce_t,
                  enew_o, ssh_o, ss_o, bn_o,
                  ir0, ir1, ir2, io0, io1, io2,
                  dh0, dh1, dh2,
                  ce0, ce1, ce2, bh0, bh1, bh2,
                  bn_v, zero_v, ssh_sh, ss_sh,
                  sx0, sx1, sx2, sg0, sg1, sg2, si0, si1, si2,
                  so0, so1, so2, ss0, ss1, ss2):
    c = lax.axis_index("c")
    s = lax.axis_index("s")
    off = c * N
    IR = (ir0, ir1, ir2)
    IO = (io0, io1, io2)
    DH = (dh0, dh1, dh2)
    CE = (ce0, ce1, ce2)
    BH = (bh0, bh1, bh2)
    SX = (sx0, sx1, sx2)
    SG = (sg0, sg1, sg2)
    SI = (si0, si1, si2)
    SO = (so0, so1, so2)
    SS = (ss0, ss1, ss2)

    def _zrow(r, _):
        for kk in range(H // L):
            zero_v[r, pl.ds(kk * L, L)] = jnp.zeros((L,), F32)
        return 0
    lax.fori_loop(0, ZR, _zrow, 0)
    for piece in range(NPT // ZR):
        pltpu.sync_copy(zero_v, ssh_sh.at[pl.ds(s * NPT + piece * ZR, ZR)])
        pltpu.sync_copy(zero_v, ss_sh.at[pl.ds(s * NPT + piece * ZR, ZR)])

    @pl.when(s == NS - 1)
    def _zero_tail():
        pltpu.sync_copy(zero_v.at[pl.ds(0, N - NS * NPT)],
                        ssh_sh.at[pl.ds(NS * NPT, N - NS * NPT)])
        pltpu.sync_copy(zero_v.at[pl.ds(0, N - NS * NPT)],
                        ss_sh.at[pl.ds(NS * NPT, N - NS * NPT)])

    plsc.subcore_barrier()

    def _rowc(t):
        return jnp.minimum(t * NS + s, NROWS - 1)

    def _issue_idx(t, q):
        pltpu.async_copy(idx_hbm.at[pl.ds(2 * _rowc(t), 2)], IR[q], SX[q])

    def _stage(t, q):
        pltpu.make_async_copy(idx_hbm.at[pl.ds(0, 2)], IR[q], SX[q]).wait()
        valid = (t * NS + s) < NROWS

        def _off(r, _):
            sl = pl.ds(r * L, L)
            sv = IR[q][0, sl]
            dv = IR[q][1, sl]
            IO[q][0, sl] = sv + off
            IO[q][1, sl] = dv + off
            IO[q][2, sl] = jnp.where(valid, dv, jnp.full((L,), N, jnp.int32))
            return 0
        lax.fori_loop(0, CH // L, _off, 0)
        rowc = _rowc(t)
        pltpu.async_copy(dh_t.at[IO[q].at[0]], DH[q], SG[q])
        pltpu.async_copy(bh_t.at[IO[q].at[0]], BH[q], SI[q])
        pltpu.async_copy(ce_t.at[pl.ds(rowc * CH, CH), pl.ds(c * H, H)],
                         CE[q], SI[q])

    def _wait_dh_start_eh(q):
        pltpu.make_async_copy(dh_t.at[IO[q].at[0]], DH[q], SG[q]).wait()
        pltpu.async_copy(eh_t.at[IO[q].at[1]], DH[q], SG[q], add=True)

    def _wait_reads(q):
        pltpu.make_async_copy(bh_t.at[IO[q].at[0]], BH[q], SI[q]).wait()
        pltpu.make_async_copy(ce_t.at[pl.ds(0, CH), pl.ds(c * H, H)],
                              CE[q], SI[q]).wait()
        pltpu.make_async_copy(eh_t.at[IO[q].at[1]], DH[q], SG[q]).wait()

    def _compute(t, q, carry):
        zvec = jnp.zeros((L,), F32)

        def _row(r, cr):
            acc = list(cr)
            for kk in range(H // L):
                sl = pl.ds(kk * L, L)
                x = DH[q][r, sl] + CE[q][r, sl]
                CE[q][r, sl] = x
                sg = 1.0 / (1.0 + jnp.exp(-x))
                DH[q][r, sl] = sg
                BH[q][r, sl] = BH[q][r, sl] * sg
                acc[kk] = acc[kk] + x
                acc[4 + kk] = acc[4 + kk] + x * x
            return tuple(acc)
        part = lax.fori_loop(0, CH, _row, (zvec,) * (2 * (H // L)))
        m = jnp.where((t * NS + s) < NROWS, 1.0, 0.0).astype(F32)
        return tuple(carry[i] + part[i] * m for i in range(2 * (H // L)))

    def _issue_outputs(t, q):
        rowc = _rowc(t)
        pltpu.async_copy(CE[q], enew_o.at[pl.ds(rowc * CH, CH),
                                          pl.ds(c * H, H)], SO[q])
        pltpu.async_copy(BH[q], ssh_sh.at[IO[q].at[2]], SS[q], add=True)
        pltpu.async_copy(DH[q], ss_sh.at[IO[q].at[2]], SS[q], add=True)

    def _drain_outputs(q):
        pltpu.make_async_copy(CE[q], enew_o.at[pl.ds(0, CH),
                                               pl.ds(c * H, H)], SO[q]).wait()
        pltpu.make_async_copy(BH[q], ssh_sh.at[IO[q].at[2]], SS[q]).wait()
        pltpu.make_async_copy(DH[q], ss_sh.at[IO[q].at[2]], SS[q]).wait()

    _issue_idx(0, 0)
    _stage(0, 0)

    def _block(jj, t, q, carry, first):
        qn = (q + 1) % NBUF
        _wait_dh_start_eh(q)
        if first:
            pl.when(jj > 0)(lambda: _drain_outputs(qn))
        else:
            _drain_outputs(qn)
        _issue_idx(t + 1, qn)
        _wait_reads(q)
        carry = _compute(t, q, carry)
        _issue_outputs(t, q)
        _stage(t + 1, qn)
        return carry

    def _fbody(jj, carry):
        t0 = jj * NBUF
        carry = _block(jj, t0, 0, carry, True)
        carry = _block(jj, t0 + 1, 1, carry, True)
        carry = _block(jj, t0 + 2, 2, carry, False)
        return carry

    zvec = jnp.zeros((L,), F32)
    carry = lax.fori_loop(0, (NBLK - 1) // NBUF, _fbody,
                          (zvec,) * (2 * (H // L)))

    t_last = NBLK - 1
    _wait_dh_start_eh(0)
    _drain_outputs(1)
    _wait_reads(0)
    carry = _compute(t_last, 0, carry)
    _issue_outputs(t_last, 0)
    _drain_outputs(2)
    _drain_outputs(0)

    def _bnz(r, _):
        for kk in range(2 * H // L):
            bn_v[r, pl.ds(kk * L, L)] = jnp.zeros((L,), F32)
        return 0
    lax.fori_loop(0, 8, _bnz, 0)
    for kk in range(H // L):
        bn_v[0, pl.ds(kk * L, L)] = carry[kk]
        bn_v[0, pl.ds(H + kk * L, L)] = carry[4 + kk]
    pltpu.sync_copy(bn_v, bn_o.at[c * NS + s])

    plsc.subcore_barrier()
    pltpu.sync_copy(ssh_sh.at[pl.ds(s * NPT, NPT)],
                    ssh_o.at[pl.ds(c * N + s * NPT, NPT)])
    pltpu.sync_copy(ss_sh.at[pl.ds(s * NPT, NPT)],
                    ss_o.at[pl.ds(c * N + s * NPT, NPT)])

    @pl.when(s == NS - 1)
    def _drain_tail():
        pltpu.sync_copy(ssh_sh.at[pl.ds(NS * NPT, N - NS * NPT)],
                        ssh_o.at[pl.ds(c * N + NS * NPT, N - NS * NPT)])
        pltpu.sync_copy(ss_sh.at[pl.ds(NS * NPT, N - NS * NPT)],
                        ss_o.at[pl.ds(c * N + NS * NPT, N - NS * NPT)])


def _sc_edge(idx_packed, dh_t, eh_t, bh_t, ce_t):
    mesh = plsc.VectorSubcoreMesh(core_axis_name="c", subcore_axis_name="s")
    fn = functools.partial(
        pl.kernel,
        mesh=mesh,
        compiler_params=pltpu.CompilerParams(use_tc_tiling_on_sc=False),
        out_type=[jax.ShapeDtypeStruct((E, D), F32),
                  jax.ShapeDtypeStruct((2 * N, H), F32),
                  jax.ShapeDtypeStruct((2 * N, H), F32),
                  jax.ShapeDtypeStruct((NC * NS, 8, 2 * H), F32)],
        scratch_types=(
            [pltpu.VMEM((2, CH), jnp.int32) for _ in range(NBUF)] +
            [pltpu.VMEM((3, CH), jnp.int32) for _ in range(NBUF)] +
            [pltpu.VMEM((CH, H), F32) for _ in range(3 * NBUF)] +
            [pltpu.VMEM((8, 2 * H), F32),
             pltpu.VMEM((ZR, H), F32),
             pltpu.VMEM_SHARED((N + 8, H), F32),
             pltpu.VMEM_SHARED((N + 8, H), F32)]
            + [pltpu.SemaphoreType.DMA for _ in range(5 * NBUF)]
        ))(_sc_edge_body)
    return fn(idx_packed, dh_t, eh_t, bh_t, ce_t)


def _eout_body(e_ref, enew_ref, bn_ref, g_ref, b_ref, out_ref):
    bn = bn_ref[...]
    s0 = jnp.sum(bn[:NS * 8, :H], axis=0)
    s1 = jnp.sum(bn[NS * 8:, :H], axis=0)
    q0 = jnp.sum(bn[:NS * 8, H:], axis=0)
    q1 = jnp.sum(bn[NS * 8:, H:], axis=0)
    mu = jnp.concatenate([s0, s1]) * (1.0 / E)
    msq = jnp.concatenate([q0, q1]) * (1.0 / E)
    var = msq - mu * mu
    inv = lax.rsqrt(var + 1e-5)
    normed = (enew_ref[...] - mu) * inv * g_ref[...] + b_ref[...]
    out_ref[...] = e_ref[...] + jnp.maximum(normed, 0.0)


def _e_out(e, enew, bn_p, gamma, beta):
    blk = 2560
    return pl.pallas_call(
        _eout_body,
        grid=(E // blk,),
        in_specs=[pl.BlockSpec((blk, D), lambda i: (i, 0)),
                  pl.BlockSpec((blk, D), lambda i: (i, 0)),
                  pl.BlockSpec((NC * NS * 8, 2 * H), lambda i: (0, 0)),
                  pl.BlockSpec((1, D), lambda i: (0, 0)),
                  pl.BlockSpec((1, D), lambda i: (0, 0))],
        out_specs=pl.BlockSpec((blk, D), lambda i: (i, 0)),
        out_shape=jax.ShapeDtypeStruct((E, D), F32),
    )(e, enew, bn_p, gamma, beta)


def _hout_body(h_ref, ah_ref, ssh_ref, ss_ref, g_ref, b_ref, out_ref):
    ssh = jnp.concatenate([ssh_ref[0], ssh_ref[1]], axis=1)
    ss = jnp.concatenate([ss_ref[0], ss_ref[1]], axis=1)
    h_new = ah_ref[...] + ssh / (ss + 1e-6)
    mu = jnp.mean(h_new, axis=0)
    var = jnp.mean(h_new * h_new, axis=0) - mu * mu
    normed = (h_new - mu) * lax.rsqrt(var + 1e-5) * g_ref[...] + b_ref[...]
    out_ref[...] = h_ref[...] + jnp.maximum(normed, 0.0)


def _h_out(h, ah, ssh, ss, gamma, beta):
    return pl.pallas_call(
        _hout_body,
        in_specs=[pl.BlockSpec((N, D), lambda: (0, 0)),
                  pl.BlockSpec((N, D), lambda: (0, 0)),
                  pl.BlockSpec((2, N, H), lambda: (0, 0, 0)),
                  pl.BlockSpec((2, N, H), lambda: (0, 0, 0)),
                  pl.BlockSpec((1, D), lambda: (0, 0)),
                  pl.BlockSpec((1, D), lambda: (0, 0))],
        out_specs=pl.BlockSpec((N, D), lambda: (0, 0)),
        out_shape=jax.ShapeDtypeStruct((N, D), F32),
    )(h, ah, ssh, ss, gamma, beta)


def kernel(h, e, edge_index, A_w, A_b, B_w, B_b, C_w, C_b, D_w, D_b, E_w, E_b,
           bn_h_gamma, bn_h_beta, bn_e_gamma, bn_e_beta):
    ah, bh_c, dh_c, eh_c = _node_linears(
        h, A_w, A_b.reshape(1, D), B_w, B_b.reshape(1, D),
        D_w, D_b.reshape(1, D), E_w, E_b.reshape(1, D))
    ce = _ce_full(e, C_w, C_b.reshape(1, D))

    idx_packed = jnp.stack(
        [edge_index[0].reshape(NROWS, CH), edge_index[1].reshape(NROWS, CH)],
        axis=1).reshape(2 * NROWS, CH)
    enew, ssh_f, ss_f, bn_p = _sc_edge(
        idx_packed,
        dh_c.reshape(2 * N, H), eh_c.reshape(2 * N, H),
        bh_c.reshape(2 * N, H), ce)

    e_out = _e_out(e, enew, bn_p.reshape(NC * NS * 8, 2 * H),
                   bn_e_gamma.reshape(1, D), bn_e_beta.reshape(1, D))
    h_out = _h_out(h, ah, ssh_f.reshape(2, N, H), ss_f.reshape(2, N, H),
                   bn_h_gamma.reshape(1, D), bn_h_beta.reshape(1, D))
    return (h_out, e_out)


# speedup vs baseline: 1.0522x; 1.0522x over previous
import functools

import jax
import jax.numpy as jnp
from jax import lax
from jax.experimental import pallas as pl
from jax.experimental.pallas import tpu as pltpu
from jax.experimental.pallas import tpu_sc as plsc

N = 10000
E = 320000
D = 128
H = 64
NC = 2
NS = 16
L = 16
CH = 80
NROWS = E // CH
NPT = 624
ZR = 16
F32 = jnp.float32


def _node_lin_body(h_ref, aw, ab, bw, bb, dw, db, ew, eb,
                   ah_o, bh_o, dh_o, eh_o):
    x = h_ref[...]
    ah_o[...] = jnp.dot(x, aw[...], preferred_element_type=F32) + ab[...]
    for w, b, o in ((bw, bb, bh_o), (dw, db, dh_o), (ew, eb, eh_o)):
        full = jnp.dot(x, w[...], preferred_element_type=F32) + b[...]
        o[0] = full[:, :H]
        o[1] = full[:, H:]


def _node_linears(h, A_w, A_b, B_w, B_b, D_w, D_b, E_w, E_b):
    blk = 2000
    grid = (N // blk,)
    wspec = pl.BlockSpec((D, D), lambda i: (0, 0))
    bspec = pl.BlockSpec((1, D), lambda i: (0, 0))
    half = pl.BlockSpec((2, blk, H), lambda i: (0, i, 0))
    return pl.pallas_call(
        _node_lin_body,
        grid=grid,
        in_specs=[pl.BlockSpec((blk, D), lambda i: (i, 0)),
                  wspec, bspec, wspec, bspec, wspec, bspec, wspec, bspec],
        out_specs=[pl.BlockSpec((blk, D), lambda i: (i, 0)), half, half, half],
        out_shape=[jax.ShapeDtypeStruct((N, D), F32),
                   jax.ShapeDtypeStruct((2, N, H), F32),
                   jax.ShapeDtypeStruct((2, N, H), F32),
                   jax.ShapeDtypeStruct((2, N, H), F32)],
    )(h, A_w, A_b, B_w, B_b, D_w, D_b, E_w, E_b)


def _ce_body(e_ref, cw, cb, ce_o):
    ce_o[...] = jnp.dot(e_ref[...], cw[...], preferred_element_type=F32) + cb[...]


def _ce_full(e, C_w, C_b):
    blk = 2560
    return pl.pallas_call(
        _ce_body,
        grid=(E // blk,),
        in_specs=[pl.BlockSpec((blk, D), lambda i: (i, 0)),
                  pl.BlockSpec((D, D), lambda i: (0, 0)),
                  pl.BlockSpec((1, D), lambda i: (0, 0))],
        out_specs=pl.BlockSpec((blk, D), lambda i: (i, 0)),
        out_shape=jax.ShapeDtypeStruct((E, D), F32),
    )(e, C_w, C_b)


NBUF = 2
NBLK = NROWS // NS + 1


def _sc_edge_body(idx_hbm, dh_t, eh_t, bh_t, ce_t,
                  enew_o, ssh_o, ss_o, bn_o,
                  ir0, ir1, io0, io1,
                  dh0, dh1, eh0, eh1,
                  ce0, ce1, bh0, bh1,
                  bn_v, zero_v, ssh_sh, ss_sh,
                  sx0, sx1, si0, si1,
                  so0, so1, ss0, ss1):
    c = lax.axis_index("c")
    s = lax.axis_index("s")
    off = c * N
    IR = (ir0, ir1)
    IO = (io0, io1)
    DH = (dh0, dh1)
    EH = (eh0, eh1)
    CE = (ce0, ce1)
    BH = (bh0, bh1)
    SX = (sx0, sx1)
    SI = (si0, si1)
    SO = (so0, so1)
    SS = (ss0, ss1)

    def _zrow(r, _):
        for kk in range(H // L):
            zero_v[r, pl.ds(kk * L, L)] = jnp.zeros((L,), F32)
        return 0
    lax.fori_loop(0, ZR, _zrow, 0)
    for piece in range(NPT // ZR):
        pltpu.sync_copy(zero_v, ssh_sh.at[pl.ds(s * NPT + piece * ZR, ZR)])
        pltpu.sync_copy(zero_v, ss_sh.at[pl.ds(s * NPT + piece * ZR, ZR)])

    @pl.when(s == NS - 1)
    def _zero_tail():
        pltpu.sync_copy(zero_v.at[pl.ds(0, N - NS * NPT)],
                        ssh_sh.at[pl.ds(NS * NPT, N - NS * NPT)])
        pltpu.sync_copy(zero_v.at[pl.ds(0, N - NS * NPT)],
                        ss_sh.at[pl.ds(NS * NPT, N - NS * NPT)])

    plsc.subcore_barrier()

    def _rowc(t):
        return jnp.minimum(t * NS + s, NROWS - 1)

    def _issue_idx(t, q):
        pltpu.async_copy(idx_hbm.at[pl.ds(2 * _rowc(t), 2)], IR[q], SX[q])

    def _stage(t, q):
        pltpu.make_async_copy(idx_hbm.at[pl.ds(0, 2)], IR[q], SX[q]).wait()
        valid = (t * NS + s) < NROWS

        def _off(r, _):
            sl = pl.ds(r * L, L)
            sv = IR[q][0, sl]
            dv = IR[q][1, sl]
            IO[q][0, sl] = sv + off
            IO[q][1, sl] = dv + off
            IO[q][2, sl] = jnp.where(valid, dv, jnp.full((L,), N, jnp.int32))
            return 0
        lax.fori_loop(0, CH // L, _off, 0)
        rowc = _rowc(t)
        pltpu.async_copy(dh_t.at[IO[q].at[0]], DH[q], SI[q])
        pltpu.async_copy(eh_t.at[IO[q].at[1]], EH[q], SI[q])
        pltpu.async_copy(bh_t.at[IO[q].at[0]], BH[q], SI[q])
        pltpu.async_copy(ce_t.at[pl.ds(rowc * CH, CH), pl.ds(c * H, H)],
                         CE[q], SI[q])

    def _wait_reads(q):
        pltpu.make_async_copy(dh_t.at[IO[q].at[0]], DH[q], SI[q]).wait()
        pltpu.make_async_copy(eh_t.at[IO[q].at[1]], EH[q], SI[q]).wait()
        pltpu.make_async_copy(bh_t.at[IO[q].at[0]], BH[q], SI[q]).wait()
        pltpu.make_async_copy(ce_t.at[pl.ds(0, CH), pl.ds(c * H, H)],
                              CE[q], SI[q]).wait()

    def _compute(t, q, carry):
        zvec = jnp.zeros((L,), F32)

        def _row(r, cr):
            acc = list(cr)
            for kk in range(H // L):
                sl = pl.ds(kk * L, L)
                x = DH[q][r, sl] + EH[q][r, sl] + CE[q][r, sl]
                CE[q][r, sl] = x
                sg = 1.0 / (1.0 + jnp.exp(-x))
                DH[q][r, sl] = sg
                BH[q][r, sl] = BH[q][r, sl] * sg
                acc[kk] = acc[kk] + x
                acc[4 + kk] = acc[4 + kk] + x * x
            return tuple(acc)
        part = lax.fori_loop(0, CH, _row, (zvec,) * (2 * (H // L)))
        m = jnp.where((t * NS + s) < NROWS, 1.0, 0.0).astype(F32)
        return tuple(carry[i] + part[i] * m for i in range(2 * (H // L)))

    def _issue_outputs(t, q):
        rowc = _rowc(t)
        pltpu.async_copy(CE[q], enew_o.at[pl.ds(rowc * CH, CH),
                                          pl.ds(c * H, H)], SO[q])
        pltpu.async_copy(BH[q], ssh_sh.at[IO[q].at[2]], SS[q], add=True)
        pltpu.async_copy(DH[q], ss_sh.at[IO[q].at[2]], SS[q], add=True)

    def _drain_outputs(q):
        pltpu.make_async_copy(CE[q], enew_o.at[pl.ds(0, CH),
                                               pl.ds(c * H, H)], SO[q]).wait()
        pltpu.make_async_copy(BH[q], ssh_sh.at[IO[q].at[2]], SS[q]).wait()
        pltpu.make_async_copy(DH[q], ss_sh.at[IO[q].at[2]], SS[q]).wait()

    _issue_idx(0, 0)
    _stage(0, 0)

    def _block(jj, t, q, carry, first):
        qn = (q + 1) % NBUF
        if first:
            pl.when(jj > 0)(lambda: _drain_outputs(qn))
        else:
            _drain_outputs(qn)
        _issue_idx(t + 1, qn)
        _wait_reads(q)
        carry = _compute(t, q, carry)
        _issue_outputs(t, q)
        _stage(t + 1, qn)
        return carry

    def _fbody(jj, carry):
        t0 = jj * NBUF
        carry = _block(jj, t0, 0, carry, True)
        carry = _block(jj, t0 + 1, 1, carry, False)
        return carry

    zvec = jnp.zeros((L,), F32)
    carry = lax.fori_loop(0, (NBLK - 1) // NBUF, _fbody,
                          (zvec,) * (2 * (H // L)))

    t_last = NBLK - 1
    _wait_reads(0)
    carry = _compute(t_last, 0, carry)
    _issue_outputs(t_last, 0)
    _drain_outputs(1)
    _drain_outputs(0)

    def _bnz(r, _):
        for kk in range(2 * H // L):
            bn_v[r, pl.ds(kk * L, L)] = jnp.zeros((L,), F32)
        return 0
    lax.fori_loop(0, 8, _bnz, 0)
    for kk in range(H // L):
        bn_v[0, pl.ds(kk * L, L)] = carry[kk]
        bn_v[0, pl.ds(H + kk * L, L)] = carry[4 + kk]
    pltpu.sync_copy(bn_v, bn_o.at[c * NS + s])

    plsc.subcore_barrier()
    pltpu.sync_copy(ssh_sh.at[pl.ds(s * NPT, NPT)],
                    ssh_o.at[pl.ds(c * N + s * NPT, NPT)])
    pltpu.sync_copy(ss_sh.at[pl.ds(s * NPT, NPT)],
                    ss_o.at[pl.ds(c * N + s * NPT, NPT)])

    @pl.when(s == NS - 1)
    def _drain_tail():
        pltpu.sync_copy(ssh_sh.at[pl.ds(NS * NPT, N - NS * NPT)],
                        ssh_o.at[pl.ds(c * N + NS * NPT, N - NS * NPT)])
        pltpu.sync_copy(ss_sh.at[pl.ds(NS * NPT, N - NS * NPT)],
                        ss_o.at[pl.ds(c * N + NS * NPT, N - NS * NPT)])


def _sc_edge(idx_packed, dh_t, eh_t, bh_t, ce_t):
    mesh = plsc.VectorSubcoreMesh(core_axis_name="c", subcore_axis_name="s")
    fn = functools.partial(
        pl.kernel,
        mesh=mesh,
        compiler_params=pltpu.CompilerParams(use_tc_tiling_on_sc=False),
        out_type=[jax.ShapeDtypeStruct((E, D), F32),
                  jax.ShapeDtypeStruct((2 * N, H), F32),
                  jax.ShapeDtypeStruct((2 * N, H), F32),
                  jax.ShapeDtypeStruct((NC * NS, 8, 2 * H), F32)],
        scratch_types=(
            [pltpu.VMEM((2, CH), jnp.int32) for _ in range(NBUF)] +
            [pltpu.VMEM((3, CH), jnp.int32) for _ in range(NBUF)] +
            [pltpu.VMEM((CH, H), F32) for _ in range(4 * NBUF)] +
            [pltpu.VMEM((8, 2 * H), F32),
             pltpu.VMEM((ZR, H), F32),
             pltpu.VMEM_SHARED((N + 8, H), F32),
             pltpu.VMEM_SHARED((N + 8, H), F32)]
            + [pltpu.SemaphoreType.DMA for _ in range(4 * NBUF)]
        ))(_sc_edge_body)
    return fn(idx_packed, dh_t, eh_t, bh_t, ce_t)


def _eout_body(e_ref, enew_ref, bn_ref, g_ref, b_ref, out_ref):
    bn = bn_ref[...]
    s0 = jnp.sum(bn[:NS * 8, :H], axis=0)
    s1 = jnp.sum(bn[NS * 8:, :H], axis=0)
    q0 = jnp.sum(bn[:NS * 8, H:], axis=0)
    q1 = jnp.sum(bn[NS * 8:, H:], axis=0)
    mu = jnp.concatenate([s0, s1]) * (1.0 / E)
    msq = jnp.concatenate([q0, q1]) * (1.0 / E)
    var = msq - mu * mu
    inv = lax.rsqrt(var + 1e-5)
    normed = (enew_ref[...] - mu) * inv * g_ref[...] + b_ref[...]
    out_ref[...] = e_ref[...] + jnp.maximum(normed, 0.0)


def _e_out(e, enew, bn_p, gamma, beta):
    blk = 2560
    return pl.pallas_call(
        _eout_body,
        grid=(E // blk,),
        in_specs=[pl.BlockSpec((blk, D), lambda i: (i, 0)),
                  pl.BlockSpec((blk, D), lambda i: (i, 0)),
                  pl.BlockSpec((NC * NS * 8, 2 * H), lambda i: (0, 0)),
                  pl.BlockSpec((1, D), lambda i: (0, 0)),
                  pl.BlockSpec((1, D), lambda i: (0, 0))],
        out_specs=pl.BlockSpec((blk, D), lambda i: (i, 0)),
        out_shape=jax.ShapeDtypeStruct((E, D), F32),
    )(e, enew, bn_p, gamma, beta)


def _hout_body(h_ref, ah_ref, ssh_ref, ss_ref, g_ref, b_ref, out_ref):
    ssh = jnp.concatenate([ssh_ref[0], ssh_ref[1]], axis=1)
    ss = jnp.concatenate([ss_ref[0], ss_ref[1]], axis=1)
    h_new = ah_ref[...] + ssh / (ss + 1e-6)
    mu = jnp.mean(h_new, axis=0)
    var = jnp.mean(h_new * h_new, axis=0) - mu * mu
    normed = (h_new - mu) * lax.rsqrt(var + 1e-5) * g_ref[...] + b_ref[...]
    out_ref[...] = h_ref[...] + jnp.maximum(normed, 0.0)


def _h_out(h, ah, ssh, ss, gamma, beta):
    return pl.pallas_call(
        _hout_body,
        in_specs=[pl.BlockSpec((N, D), lambda: (0, 0)),
                  pl.BlockSpec((N, D), lambda: (0, 0)),
                  pl.BlockSpec((2, N, H), lambda: (0, 0, 0)),
                  pl.BlockSpec((2, N, H), lambda: (0, 0, 0)),
                  pl.BlockSpec((1, D), lambda: (0, 0)),
                  pl.BlockSpec((1, D), lambda: (0, 0))],
        out_specs=pl.BlockSpec((N, D), lambda: (0, 0)),
        out_shape=jax.ShapeDtypeStruct((N, D), F32),
    )(h, ah, ssh, ss, gamma, beta)


def kernel(h, e, edge_index, A_w, A_b, B_w, B_b, C_w, C_b, D_w, D_b, E_w, E_b,
           bn_h_gamma, bn_h_beta, bn_e_gamma, bn_e_beta):
    ah, bh_c, dh_c, eh_c = _node_linears(
        h, A_w, A_b.reshape(1, D), B_w, B_b.reshape(1, D),
        D_w, D_b.reshape(1, D), E_w, E_b.reshape(1, D))
    ce = _ce_full(e, C_w, C_b.reshape(1, D))

    idx_packed = jnp.stack(
        [edge_index[0].reshape(NROWS, CH), edge_index[1].reshape(NROWS, CH)],
        axis=1).reshape(2 * NROWS, CH)
    enew, ssh_f, ss_f, bn_p = _sc_edge(
        idx_packed,
        dh_c.reshape(2 * N, H), eh_c.reshape(2 * N, H),
        bh_c.reshape(2 * N, H), ce)

    e_out = _e_out(e, enew, bn_p.reshape(NC * NS * 8, 2 * H),
                   bn_e_gamma.reshape(1, D), bn_e_beta.reshape(1, D))
    h_out = _h_out(h, ah, ssh_f.reshape(2, N, H), ss_f.reshape(2, N, H),
                   bn_h_gamma.reshape(1, D), bn_h_beta.reshape(1, D))
    return (h_out, e_out)


# speedup vs baseline: 1.2378x; 1.1764x over previous
import functools

import jax
import jax.numpy as jnp
from jax import lax
from jax.experimental import pallas as pl
from jax.experimental.pallas import tpu as pltpu
from jax.experimental.pallas import tpu_sc as plsc

N = 10000
E = 320000
D = 128
H = 64
NC = 2
NS = 16
L = 16
CH = 80
NROWS = E // CH
NPT = 624
ZR = 16
F32 = jnp.float32


def _node_lin_body(h_ref, aw, ab, bw, bb, dw, db, ew, eb,
                   ah_o, bh_o, dh_o, eh_o):
    x = h_ref[...]
    ah_o[...] = jnp.dot(x, aw[...], preferred_element_type=F32) + ab[...]
    for w, b, o in ((bw, bb, bh_o), (dw, db, dh_o), (ew, eb, eh_o)):
        full = jnp.dot(x, w[...], preferred_element_type=F32) + b[...]
        o[0] = full[:, :H]
        o[1] = full[:, H:]


def _node_linears(h, A_w, A_b, B_w, B_b, D_w, D_b, E_w, E_b):
    blk = 2000
    grid = (N // blk,)
    wspec = pl.BlockSpec((D, D), lambda i: (0, 0))
    bspec = pl.BlockSpec((1, D), lambda i: (0, 0))
    half = pl.BlockSpec((2, blk, H), lambda i: (0, i, 0))
    return pl.pallas_call(
        _node_lin_body,
        grid=grid,
        in_specs=[pl.BlockSpec((blk, D), lambda i: (i, 0)),
                  wspec, bspec, wspec, bspec, wspec, bspec, wspec, bspec],
        out_specs=[pl.BlockSpec((blk, D), lambda i: (i, 0)), half, half, half],
        out_shape=[jax.ShapeDtypeStruct((N, D), F32),
                   jax.ShapeDtypeStruct((2, N, H), F32),
                   jax.ShapeDtypeStruct((2, N, H), F32),
                   jax.ShapeDtypeStruct((2, N, H), F32)],
    )(h, A_w, A_b, B_w, B_b, D_w, D_b, E_w, E_b)


def _ce_body(e_ref, cw, cb, ce_o):
    ce_o[...] = jnp.dot(e_ref[...], cw[...], preferred_element_type=F32) + cb[...]


def _ce_full(e, C_w, C_b):
    blk = 4000
    return pl.pallas_call(
        _ce_body,
        grid=(E // blk,),
        in_specs=[pl.BlockSpec((blk, D), lambda i: (i, 0)),
                  pl.BlockSpec((D, D), lambda i: (0, 0)),
                  pl.BlockSpec((1, D), lambda i: (0, 0))],
        out_specs=pl.BlockSpec((blk, D), lambda i: (i, 0)),
        out_shape=jax.ShapeDtypeStruct((E, D), F32),
    )(e, C_w, C_b)


NBUF = 2
NBLK = NROWS // NS + 1


def _sc_edge_body(idx_hbm, dh_t, eh_t, bh_t, ce_t,
                  enew_o, ssh_o, ss_o, bn_o,
                  ir0, ir1, io0, io1,
                  dh0, dh1, eh0, eh1,
                  ce0, ce1, bh0, bh1,
                  bn_v, zero_v, ssh_sh, ss_sh,
                  sx0, sx1, si0, si1,
                  so0, so1, ss0, ss1):
    c = lax.axis_index("c")
    s = lax.axis_index("s")
    off = c * N
    IR = (ir0, ir1)
    IO = (io0, io1)
    DH = (dh0, dh1)
    EH = (eh0, eh1)
    CE = (ce0, ce1)
    BH = (bh0, bh1)
    SX = (sx0, sx1)
    SI = (si0, si1)
    SO = (so0, so1)
    SS = (ss0, ss1)

    def _zrow(r, _):
        for kk in range(H // L):
            zero_v[r, pl.ds(kk * L, L)] = jnp.zeros((L,), F32)
        return 0
    lax.fori_loop(0, ZR, _zrow, 0)
    for piece in range(NPT // ZR):
        pltpu.sync_copy(zero_v, ssh_sh.at[pl.ds(s * NPT + piece * ZR, ZR)])
        pltpu.sync_copy(zero_v, ss_sh.at[pl.ds(s * NPT + piece * ZR, ZR)])

    @pl.when(s == NS - 1)
    def _zero_tail():
        pltpu.sync_copy(zero_v.at[pl.ds(0, N - NS * NPT)],
                        ssh_sh.at[pl.ds(NS * NPT, N - NS * NPT)])
        pltpu.sync_copy(zero_v.at[pl.ds(0, N - NS * NPT)],
                        ss_sh.at[pl.ds(NS * NPT, N - NS * NPT)])

    plsc.subcore_barrier()

    def _rowc(t):
        return jnp.minimum(t * NS + s, NROWS - 1)

    def _issue_idx(t, q):
        pltpu.async_copy(idx_hbm.at[pl.ds(2 * _rowc(t), 2)], IR[q], SX[q])

    def _stage(t, q):
        pltpu.make_async_copy(idx_hbm.at[pl.ds(0, 2)], IR[q], SX[q]).wait()
        valid = (t * NS + s) < NROWS

        def _off(r, _):
            sl = pl.ds(r * L, L)
            sv = IR[q][0, sl]
            dv = IR[q][1, sl]
            IO[q][0, sl] = sv + off
            IO[q][1, sl] = dv + off
            IO[q][2, sl] = jnp.where(valid, dv, jnp.full((L,), N, jnp.int32))
            return 0
        lax.fori_loop(0, CH // L, _off, 0)
        rowc = _rowc(t)
        pltpu.async_copy(dh_t.at[IO[q].at[0]], DH[q], SI[q])
        pltpu.async_copy(eh_t.at[IO[q].at[1]], EH[q], SI[q])
        pltpu.async_copy(bh_t.at[IO[q].at[0]], BH[q], SI[q])
        pltpu.async_copy(ce_t.at[pl.ds(rowc * CH, CH), pl.ds(c * H, H)],
                         CE[q], SI[q])

    def _wait_reads(q):
        pltpu.make_async_copy(dh_t.at[IO[q].at[0]], DH[q], SI[q]).wait()
        pltpu.make_async_copy(eh_t.at[IO[q].at[1]], EH[q], SI[q]).wait()
        pltpu.make_async_copy(bh_t.at[IO[q].at[0]], BH[q], SI[q]).wait()
        pltpu.make_async_copy(ce_t.at[pl.ds(0, CH), pl.ds(c * H, H)],
                              CE[q], SI[q]).wait()

    def _compute(t, q, carry):
        zvec = jnp.zeros((L,), F32)

        def _row(r, cr):
            acc = list(cr)
            for kk in range(H // L):
                sl = pl.ds(kk * L, L)
                x = DH[q][r, sl] + EH[q][r, sl] + CE[q][r, sl]
                CE[q][r, sl] = x
                sg = 1.0 / (1.0 + jnp.exp(-x))
                DH[q][r, sl] = sg
                BH[q][r, sl] = BH[q][r, sl] * sg
                acc[kk] = acc[kk] + x
                acc[4 + kk] = acc[4 + kk] + x * x
            return tuple(acc)
        part = lax.fori_loop(0, CH, _row, (zvec,) * (2 * (H // L)))
        m = jnp.where((t * NS + s) < NROWS, 1.0, 0.0).astype(F32)
        return tuple(carry[i] + part[i] * m for i in range(2 * (H // L)))

    def _issue_outputs(t, q):
        rowc = _rowc(t)
        pltpu.async_copy(CE[q], enew_o.at[pl.ds(rowc * CH, CH),
                                          pl.ds(c * H, H)], SO[q])
        pltpu.async_copy(BH[q], ssh_sh.at[IO[q].at[2]], SS[q], add=True)
        pltpu.async_copy(DH[q], ss_sh.at[IO[q].at[2]], SS[q], add=True)

    def _drain_outputs(q):
        pltpu.make_async_copy(CE[q], enew_o.at[pl.ds(0, CH),
                                               pl.ds(c * H, H)], SO[q]).wait()
        pltpu.make_async_copy(BH[q], ssh_sh.at[IO[q].at[2]], SS[q]).wait()
        pltpu.make_async_copy(DH[q], ss_sh.at[IO[q].at[2]], SS[q]).wait()

    _issue_idx(0, 0)
    _stage(0, 0)

    def _block(jj, t, q, carry, first):
        qn = (q + 1) % NBUF
        if first:
            pl.when(jj > 0)(lambda: _drain_outputs(qn))
        else:
            _drain_outputs(qn)
        _issue_idx(t + 1, qn)
        _wait_reads(q)
        carry = _compute(t, q, carry)
        _issue_outputs(t, q)
        _stage(t + 1, qn)
        return carry

    def _fbody(jj, carry):
        t0 = jj * NBUF
        carry = _block(jj, t0, 0, carry, True)
        carry = _block(jj, t0 + 1, 1, carry, False)
        return carry

    zvec = jnp.zeros((L,), F32)
    carry = lax.fori_loop(0, (NBLK - 1) // NBUF, _fbody,
                          (zvec,) * (2 * (H // L)))

    t_last = NBLK - 1
    _wait_reads(0)
    carry = _compute(t_last, 0, carry)
    _issue_outputs(t_last, 0)
    _drain_outputs(1)
    _drain_outputs(0)

    def _bnz(r, _):
        for kk in range(2 * H // L):
            bn_v[r, pl.ds(kk * L, L)] = jnp.zeros((L,), F32)
        return 0
    lax.fori_loop(0, 8, _bnz, 0)
    for kk in range(H // L):
        bn_v[0, pl.ds(kk * L, L)] = carry[kk]
        bn_v[0, pl.ds(H + kk * L, L)] = carry[4 + kk]
    pltpu.sync_copy(bn_v, bn_o.at[c * NS + s])

    plsc.subcore_barrier()
    pltpu.sync_copy(ssh_sh.at[pl.ds(s * NPT, NPT)],
                    ssh_o.at[pl.ds(c * N + s * NPT, NPT)])
    pltpu.sync_copy(ss_sh.at[pl.ds(s * NPT, NPT)],
                    ss_o.at[pl.ds(c * N + s * NPT, NPT)])

    @pl.when(s == NS - 1)
    def _drain_tail():
        pltpu.sync_copy(ssh_sh.at[pl.ds(NS * NPT, N - NS * NPT)],
                        ssh_o.at[pl.ds(c * N + NS * NPT, N - NS * NPT)])
        pltpu.sync_copy(ss_sh.at[pl.ds(NS * NPT, N - NS * NPT)],
                        ss_o.at[pl.ds(c * N + NS * NPT, N - NS * NPT)])


def _sc_edge(idx_packed, dh_t, eh_t, bh_t, ce_t):
    mesh = plsc.VectorSubcoreMesh(core_axis_name="c", subcore_axis_name="s")
    fn = functools.partial(
        pl.kernel,
        mesh=mesh,
        compiler_params=pltpu.CompilerParams(use_tc_tiling_on_sc=False),
        out_type=[jax.ShapeDtypeStruct((E, D), F32),
                  jax.ShapeDtypeStruct((2 * N, H), F32),
                  jax.ShapeDtypeStruct((2 * N, H), F32),
                  jax.ShapeDtypeStruct((NC * NS, 8, 2 * H), F32)],
        scratch_types=(
            [pltpu.VMEM((2, CH), jnp.int32) for _ in range(NBUF)] +
            [pltpu.VMEM((3, CH), jnp.int32) for _ in range(NBUF)] +
            [pltpu.VMEM((CH, H), F32) for _ in range(4 * NBUF)] +
            [pltpu.VMEM((8, 2 * H), F32),
             pltpu.VMEM((ZR, H), F32),
             pltpu.VMEM_SHARED((N + 8, H), F32),
             pltpu.VMEM_SHARED((N + 8, H), F32)]
            + [pltpu.SemaphoreType.DMA for _ in range(4 * NBUF)]
        ))(_sc_edge_body)
    return fn(idx_packed, dh_t, eh_t, bh_t, ce_t)


def _eout_body(e_ref, enew_ref, bn_ref, g_ref, b_ref, out_ref):
    bn = bn_ref[...]
    s0 = jnp.sum(bn[:NS * 8, :H], axis=0)
    s1 = jnp.sum(bn[NS * 8:, :H], axis=0)
    q0 = jnp.sum(bn[:NS * 8, H:], axis=0)
    q1 = jnp.sum(bn[NS * 8:, H:], axis=0)
    mu = jnp.concatenate([s0, s1]) * (1.0 / E)
    msq = jnp.concatenate([q0, q1]) * (1.0 / E)
    var = msq - mu * mu
    inv = lax.rsqrt(var + 1e-5)
    normed = (enew_ref[...] - mu) * inv * g_ref[...] + b_ref[...]
    out_ref[...] = e_ref[...] + jnp.maximum(normed, 0.0)


def _e_out(e, enew, bn_p, gamma, beta):
    blk = 4000
    return pl.pallas_call(
        _eout_body,
        grid=(E // blk,),
        in_specs=[pl.BlockSpec((blk, D), lambda i: (i, 0)),
                  pl.BlockSpec((blk, D), lambda i: (i, 0)),
                  pl.BlockSpec((NC * NS * 8, 2 * H), lambda i: (0, 0)),
                  pl.BlockSpec((1, D), lambda i: (0, 0)),
                  pl.BlockSpec((1, D), lambda i: (0, 0))],
        out_specs=pl.BlockSpec((blk, D), lambda i: (i, 0)),
        out_shape=jax.ShapeDtypeStruct((E, D), F32),
    )(e, enew, bn_p, gamma, beta)


def _hout_body(h_ref, ah_ref, ssh_ref, ss_ref, g_ref, b_ref, out_ref):
    ssh = jnp.concatenate([ssh_ref[0], ssh_ref[1]], axis=1)
    ss = jnp.concatenate([ss_ref[0], ss_ref[1]], axis=1)
    h_new = ah_ref[...] + ssh / (ss + 1e-6)
    mu = jnp.mean(h_new, axis=0)
    var = jnp.mean(h_new * h_new, axis=0) - mu * mu
    normed = (h_new - mu) * lax.rsqrt(var + 1e-5) * g_ref[...] + b_ref[...]
    out_ref[...] = h_ref[...] + jnp.maximum(normed, 0.0)


def _h_out(h, ah, ssh, ss, gamma, beta):
    return pl.pallas_call(
        _hout_body,
        in_specs=[pl.BlockSpec((N, D), lambda: (0, 0)),
                  pl.BlockSpec((N, D), lambda: (0, 0)),
                  pl.BlockSpec((2, N, H), lambda: (0, 0, 0)),
                  pl.BlockSpec((2, N, H), lambda: (0, 0, 0)),
                  pl.BlockSpec((1, D), lambda: (0, 0)),
                  pl.BlockSpec((1, D), lambda: (0, 0))],
        out_specs=pl.BlockSpec((N, D), lambda: (0, 0)),
        out_shape=jax.ShapeDtypeStruct((N, D), F32),
    )(h, ah, ssh, ss, gamma, beta)


def kernel(h, e, edge_index, A_w, A_b, B_w, B_b, C_w, C_b, D_w, D_b, E_w, E_b,
           bn_h_gamma, bn_h_beta, bn_e_gamma, bn_e_beta):
    ah, bh_c, dh_c, eh_c = _node_linears(
        h, A_w, A_b.reshape(1, D), B_w, B_b.reshape(1, D),
        D_w, D_b.reshape(1, D), E_w, E_b.reshape(1, D))
    ce = _ce_full(e, C_w, C_b.reshape(1, D))

    idx_packed = jnp.stack(
        [edge_index[0].reshape(NROWS, CH), edge_index[1].reshape(NROWS, CH)],
        axis=1).reshape(2 * NROWS, CH)
    enew, ssh_f, ss_f, bn_p = _sc_edge(
        idx_packed,
        dh_c.reshape(2 * N, H), eh_c.reshape(2 * N, H),
        bh_c.reshape(2 * N, H), ce)

    e_out = _e_out(e, enew, bn_p.reshape(NC * NS * 8, 2 * H),
                   bn_e_gamma.reshape(1, D), bn_e_beta.reshape(1, D))
    h_out = _h_out(h, ah, ssh_f.reshape(2, N, H), ss_f.reshape(2, N, H),
                   bn_h_gamma.reshape(1, D), bn_h_beta.reshape(1, D))
    return (h_out, e_out)
